# baseline (speedup 1.0000x reference)
import functools

import jax
import jax.numpy as jnp
from jax import lax
from jax.experimental import pallas as pl
from jax.experimental.pallas import tpu as pltpu
from jax.experimental.pallas import tpu_sc as plsc

N = 10000
D = 128
OUT = 128
E = 160000

NC = 2
NS = 16
K = 128
G = 8
NCH = 80
NG = NCH // G
EPT_PAD = NCH * K
N_PAD = 10112
RT = N_PAD // NS
DA = D + 16
_BLOCKS = [(o, min(K, RT - o)) for o in range(0, RT, K)]

_mesh = plsc.VectorSubcoreMesh(
    core_axis_name="c", subcore_axis_name="s", num_cores=NC, num_subcores=NS
)


@functools.partial(
    pl.kernel,
    out_type=jax.ShapeDtypeStruct((NC, N_PAD, DA), jnp.float32),
    mesh=_mesh,
    compiler_params=pltpu.CompilerParams(use_tc_tiling_on_sc=False),
    scratch_types=[
        pltpu.VMEM((2 * G, K), jnp.int32),
        pltpu.VMEM((G, K), jnp.int32),
        pltpu.VMEM((K, DA), jnp.float32),
        pltpu.VMEM((K, DA), jnp.float32),
        pltpu.SemaphoreType.DMA,
        pltpu.SemaphoreType.DMA,
        pltpu.SemaphoreType.DMA,
        pltpu.SemaphoreType.DMA,
        pltpu.SemaphoreType.DMA,
        pltpu.SemaphoreType.DMA,
        pltpu.VMEM_SHARED((N_PAD, DA), jnp.float32),
    ],
)
def _sc_agg(feat_h, srcs_h, dsts_h, seg_h, src_v, dst_v, rows0_v, rows1_v,
            sem0a, sem0b, sem1a, sem1b, semi, sems_s, acc_sh):
    c = lax.axis_index("c")
    s = lax.axis_index("s")

    zero = jnp.zeros((16,), jnp.float32)
    base = s * RT

    def fill_zero(r, _):
        for k in range(DA // 16):
            rows0_v[r, pl.ds(k * 16, 16)] = zero
        return 0

    lax.fori_loop(0, K, fill_zero, 0)

    for off, sz in _BLOCKS:
        pltpu.sync_copy(rows0_v.at[pl.ds(0, sz)],
                        acc_sh.at[pl.ds(base + off, sz)])

    plsc.subcore_barrier()

    bufs = (rows0_v, rows1_v)
    sems = ((sem0a, sem0b), (sem1a, sem1b))
    H = K // 2

    def _issue_gather(r, buf, sem2):
        pltpu.async_copy(feat_h.at[src_v.at[r, pl.ds(0, H)]],
                         buf.at[pl.ds(0, H)], sem2[0])
        pltpu.async_copy(feat_h.at[src_v.at[r, pl.ds(H, H)]],
                         buf.at[pl.ds(H, H)], sem2[1])

    def _wait_gather(r, buf, sem2):
        pltpu.make_async_copy(feat_h.at[src_v.at[r, pl.ds(0, H)]],
                              buf.at[pl.ds(0, H)], sem2[0]).wait()
        pltpu.make_async_copy(feat_h.at[src_v.at[r, pl.ds(H, H)]],
                              buf.at[pl.ds(H, H)], sem2[1]).wait()

    pltpu.sync_copy(srcs_h.at[c, s, pl.ds(0, G)], src_v.at[pl.ds(0, G)])
    _issue_gather(0, rows0_v, sems[0])

    def group(h, _):
        half = lax.rem(h, 2) * G
        nhalf = G - half

        for cj in range(G):
            b = cj % 2
            if cj == 0:
                @pl.when(h > 0)
                def _():
                    pltpu.make_async_copy(seg_h.at[c, pl.ds(0, K)],
                                          bufs[1 - b], sems_s).wait()
                pltpu.sync_copy(dsts_h.at[c, s, pl.ds(h * G, G)], dst_v)

                @pl.when(h < NG - 1)
                def _():
                    pltpu.async_copy(srcs_h.at[c, s, pl.ds((h + 1) * G, G)],
                                     src_v.at[pl.ds(nhalf, G)], semi)
            else:
                pltpu.make_async_copy(seg_h.at[c, pl.ds(0, K)],
                                      bufs[1 - b], sems_s).wait()

            if cj == G - 2:
                @pl.when(h < NG - 1)
                def _():
                    pltpu.make_async_copy(
                        srcs_h.at[c, s, pl.ds((h + 1) * G, G)],
                        src_v.at[pl.ds(nhalf, G)], semi).wait()

            if cj < G - 1:
                _issue_gather(half + cj + 1, bufs[1 - b], sems[1 - b])
            else:
                @pl.when(h < NG - 1)
                def _():
                    _issue_gather(nhalf, bufs[1 - b], sems[1 - b])

            _wait_gather(half + cj, bufs[b], sems[b])
            pltpu.async_copy(bufs[b], acc_sh.at[dst_v.at[cj]], sems_s,
                             add=True)

        return 0

    lax.fori_loop(0, NG, group, 0)
    pltpu.make_async_copy(seg_h.at[c, pl.ds(0, K)], rows0_v, sems_s).wait()

    plsc.subcore_barrier()

    for off, sz in _BLOCKS:
        pltpu.sync_copy(acc_sh.at[pl.ds(base + off, sz)],
                        seg_h.at[c, pl.ds(base + off, sz)])


BN = 1000


def _dense_body(f_ref, seg_ref, ws_ref, wn_ref, b_ref, o_ref):
    su = seg_ref[0, :, 0:D]
    sr = seg_ref[1, :, 0:D]
    cu = jnp.maximum(seg_ref[0, :, D:D + 1], 1.0)
    cr = jnp.maximum(seg_ref[1, :, D:D + 1], 1.0)
    acc = jnp.dot(f_ref[...], ws_ref[...], preferred_element_type=jnp.float32)
    acc += jnp.dot(su / cu, wn_ref[0:D, :], preferred_element_type=jnp.float32)
    acc += jnp.dot(sr / cr, wn_ref[D:2 * D, :],
                   preferred_element_type=jnp.float32)
    o_ref[...] = acc + b_ref[...]


def _dense(feat, seg, W_self, W_neigh, bias2d):
    return pl.pallas_call(
        _dense_body,
        grid=(N // BN,),
        in_specs=[
            pl.BlockSpec((BN, D), lambda i: (i, 0)),
            pl.BlockSpec((NC, BN, DA), lambda i: (0, i, 0)),
            pl.BlockSpec((D, OUT), lambda i: (0, 0)),
            pl.BlockSpec((2 * D, OUT), lambda i: (0, 0)),
            pl.BlockSpec((1, OUT), lambda i: (0, 0)),
        ],
        out_specs=pl.BlockSpec((BN, OUT), lambda i: (i, 0)),
        out_shape=jax.ShapeDtypeStruct((N, OUT), jnp.float32),
    )(feat, seg, W_self, W_neigh, bias2d)


def _prep_edges(ei):
    src = ei[0].astype(jnp.int32)
    dst = ei[1].astype(jnp.int32)
    pad = NS * EPT_PAD - E
    src = jnp.concatenate([src, jnp.zeros((pad,), jnp.int32)])
    dst = jnp.concatenate([dst, jnp.full((pad,), N, jnp.int32)])
    return src.reshape(NS, NCH, K), dst.reshape(NS, NCH, K)


@jax.jit
def kernel(feat, edge_index_user, edge_index_reuse, W_self, W_neigh, bias):
    feat = feat.astype(jnp.float32)
    aug = jnp.concatenate(
        [feat,
         jnp.ones((N, 1), jnp.float32),
         jnp.zeros((N, DA - D - 1), jnp.float32)], axis=1)
    su, du = _prep_edges(edge_index_user)
    sr, dr = _prep_edges(edge_index_reuse)
    srcs = jnp.stack([su, sr])
    dsts = jnp.stack([du, dr])
    seg = _sc_agg(aug, srcs, dsts)
    return _dense(feat, seg, W_self, W_neigh, bias.reshape(1, OUT))


# speedup vs baseline: 1.0093x; 1.0093x over previous
import functools

import jax
import jax.numpy as jnp
from jax import lax
from jax.experimental import pallas as pl
from jax.experimental.pallas import tpu as pltpu
from jax.experimental.pallas import tpu_sc as plsc

N = 10000
D = 128
OUT = 128
E = 160000

NC = 2
NS = 16
K = 128
G = 8
NCH = 80
NG = NCH // G
EPT_PAD = NCH * K
N_PAD = 10112
RT = N_PAD // NS
DA = D + 16
_BLOCKS = [(o, min(K, RT - o)) for o in range(0, RT, K)]

_mesh = plsc.VectorSubcoreMesh(
    core_axis_name="c", subcore_axis_name="s", num_cores=NC, num_subcores=NS
)


@functools.partial(
    pl.kernel,
    out_type=jax.ShapeDtypeStruct((NC, N_PAD, DA), jnp.float32),
    mesh=_mesh,
    compiler_params=pltpu.CompilerParams(use_tc_tiling_on_sc=False),
    scratch_types=[
        pltpu.VMEM((2 * G, K), jnp.int32),
        pltpu.VMEM((G, K), jnp.int32),
        pltpu.VMEM((K, DA), jnp.float32),
        pltpu.VMEM((K, DA), jnp.float32),
        pltpu.SemaphoreType.DMA,
        pltpu.SemaphoreType.DMA,
        pltpu.SemaphoreType.DMA,
        pltpu.VMEM_SHARED((N_PAD, DA), jnp.float32),
    ],
)
def _sc_agg(feat_h, srcs_h, dsts_h, seg_h, src_v, dst_v, rows0_v, rows1_v,
            sem0, sem1, semi, acc_sh):
    c = lax.axis_index("c")
    s = lax.axis_index("s")

    zero = jnp.zeros((16,), jnp.float32)
    base = s * RT

    def fill_zero(r, _):
        for k in range(DA // 16):
            rows0_v[r, pl.ds(k * 16, 16)] = zero
        return 0

    lax.fori_loop(0, K, fill_zero, 0)

    for off, sz in _BLOCKS:
        pltpu.sync_copy(rows0_v.at[pl.ds(0, sz)],
                        acc_sh.at[pl.ds(base + off, sz)])

    plsc.subcore_barrier()

    bufs = (rows0_v, rows1_v)
    sems = (sem0, sem1)

    pltpu.sync_copy(srcs_h.at[c, s, pl.ds(0, G)], src_v.at[pl.ds(0, G)])
    pltpu.async_copy(feat_h.at[src_v.at[0]], rows0_v, sem0)
    pltpu.async_copy(feat_h.at[src_v.at[1]], rows1_v, sem1)

    def group(h, _):
        half = lax.rem(h, 2) * G
        nhalf = G - half
        pltpu.sync_copy(dsts_h.at[c, s, pl.ds(h * G, G)], dst_v)

        for cj in range(G):
            b = cj % 2
            if cj == 0:
                @pl.when(h < NG - 1)
                def _():
                    pltpu.async_copy(srcs_h.at[c, s, pl.ds((h + 1) * G, G)],
                                     src_v.at[pl.ds(nhalf, G)], semi)

            if cj == G - 2:
                @pl.when(h < NG - 1)
                def _():
                    pltpu.make_async_copy(
                        srcs_h.at[c, s, pl.ds((h + 1) * G, G)],
                        src_v.at[pl.ds(nhalf, G)], semi).wait()

            pltpu.make_async_copy(feat_h.at[src_v.at[half + cj]], bufs[b],
                                  sems[b]).wait()
            pltpu.sync_copy(bufs[b], acc_sh.at[dst_v.at[cj]], add=True)

            if cj < G - 2:
                pltpu.async_copy(feat_h.at[src_v.at[half + cj + 2]],
                                 bufs[b], sems[b])
            else:
                @pl.when(h < NG - 1)
                def _():
                    pltpu.async_copy(feat_h.at[src_v.at[nhalf + cj - (G - 2)]],
                                     bufs[b], sems[b])

        return 0

    lax.fori_loop(0, NG, group, 0)

    plsc.subcore_barrier()

    for off, sz in _BLOCKS:
        pltpu.sync_copy(acc_sh.at[pl.ds(base + off, sz)],
                        seg_h.at[c, pl.ds(base + off, sz)])


BN = 1000


def _dense_body(f_ref, seg_ref, ws_ref, wn_ref, b_ref, o_ref):
    su = seg_ref[0, :, 0:D]
    sr = seg_ref[1, :, 0:D]
    cu = jnp.maximum(seg_ref[0, :, D:D + 1], 1.0)
    cr = jnp.maximum(seg_ref[1, :, D:D + 1], 1.0)
    acc = jnp.dot(f_ref[...], ws_ref[...], preferred_element_type=jnp.float32)
    acc += jnp.dot(su / cu, wn_ref[0:D, :], preferred_element_type=jnp.float32)
    acc += jnp.dot(sr / cr, wn_ref[D:2 * D, :],
                   preferred_element_type=jnp.float32)
    o_ref[...] = acc + b_ref[...]


def _dense(feat, seg, W_self, W_neigh, bias2d):
    return pl.pallas_call(
        _dense_body,
        grid=(N // BN,),
        in_specs=[
            pl.BlockSpec((BN, D), lambda i: (i, 0)),
            pl.BlockSpec((NC, BN, DA), lambda i: (0, i, 0)),
            pl.BlockSpec((D, OUT), lambda i: (0, 0)),
            pl.BlockSpec((2 * D, OUT), lambda i: (0, 0)),
            pl.BlockSpec((1, OUT), lambda i: (0, 0)),
        ],
        out_specs=pl.BlockSpec((BN, OUT), lambda i: (i, 0)),
        out_shape=jax.ShapeDtypeStruct((N, OUT), jnp.float32),
    )(feat, seg, W_self, W_neigh, bias2d)


def _prep_edges(ei):
    src = ei[0].astype(jnp.int32)
    dst = ei[1].astype(jnp.int32)
    pad = NS * EPT_PAD - E
    src = jnp.concatenate([src, jnp.zeros((pad,), jnp.int32)])
    dst = jnp.concatenate([dst, jnp.full((pad,), N, jnp.int32)])
    return src.reshape(NS, NCH, K), dst.reshape(NS, NCH, K)


@jax.jit
def kernel(feat, edge_index_user, edge_index_reuse, W_self, W_neigh, bias):
    feat = feat.astype(jnp.float32)
    aug = jnp.concatenate(
        [feat,
         jnp.ones((N, 1), jnp.float32),
         jnp.zeros((N, DA - D - 1), jnp.float32)], axis=1)
    su, du = _prep_edges(edge_index_user)
    sr, dr = _prep_edges(edge_index_reuse)
    srcs = jnp.stack([su, sr])
    dsts = jnp.stack([du, dr])
    seg = _sc_agg(aug, srcs, dsts)
    return _dense(feat, seg, W_self, W_neigh, bias.reshape(1, OUT))
